# Initial kernel scaffold; baseline (speedup 1.0000x reference)
import jax
import jax.numpy as jnp
from jax.experimental import pallas as pl


def kernel(x, edge_index, W1, b1, W2, b2, W3, b3, Wm1, bm1, Wm2, bm2):
    raise NotImplementedError("write your pallas kernel here")


# speedup vs baseline: 5.1610x; 5.1610x over previous
import functools

import jax
import jax.numpy as jnp
from jax import lax
from jax.experimental import pallas as pl
from jax.experimental.pallas import tpu as pltpu
from jax.experimental.pallas import tpu_sc as plsc

_NC = 2
_NS = 16


def _make_aggregate(N, H, E, with_deg, C=80):
    NW = _NC * _NS
    EP = E // NW
    NCH = EP // C
    assert EP * NW == E and NCH * C == EP and C % 8 == 0
    RP = (-(-N // _NS) + 7) // 8 * 8
    RP_LAST = N - (_NS - 1) * RP
    assert RP_LAST > 0 and RP_LAST % 8 == 0

    mesh = plsc.VectorSubcoreMesh(core_axis_name="c", subcore_axis_name="s")

    out_type = [jax.ShapeDtypeStruct((_NC, N, H), jnp.float32)]
    scratch = [
        pltpu.VMEM((C,), jnp.int32),
        pltpu.VMEM((C,), jnp.int32),
        pltpu.VMEM((C, H), jnp.float32),
        pltpu.VMEM_SHARED((N, H), jnp.float32),
        pltpu.SemaphoreType.DMA,
    ]
    if with_deg:
        out_type.append(jax.ShapeDtypeStruct((_NC, N), jnp.float32))
        scratch += [
            pltpu.VMEM((C,), jnp.float32),
            pltpu.VMEM_SHARED((N,), jnp.float32),
        ]

    def body(*refs):
        if with_deg:
            (m_hbm, src_hbm, dst_hbm, znh_hbm, zn_hbm,
             agg_hbm, deg_hbm, src_v, dst_v, rows_v, acc, sem,
             ones_v, dacc) = refs
        else:
            (m_hbm, src_hbm, dst_hbm, znh_hbm,
             agg_hbm, src_v, dst_v, rows_v, acc, sem) = refs
        cid = lax.axis_index("c")
        sid = lax.axis_index("s")
        wid = sid * _NC + cid

        r0 = pl.multiple_of(sid * RP, 8)

        @pl.when(sid < _NS - 1)
        def _():
            pltpu.sync_copy(znh_hbm.at[pl.ds(r0, RP)], acc.at[pl.ds(r0, RP)])

        @pl.when(sid == _NS - 1)
        def _():
            t0 = (_NS - 1) * RP
            pltpu.sync_copy(
                znh_hbm.at[pl.ds(t0, RP_LAST)], acc.at[pl.ds(t0, RP_LAST)]
            )
        if with_deg:

            @pl.when(sid == 0)
            def _():
                pltpu.sync_copy(zn_hbm, dacc)

            for j in range(C // 16):
                ones_v[pl.ds(j * 16, 16)] = jnp.ones((16,), jnp.float32)
        plsc.subcore_barrier()

        def step(i, carry):
            base = pl.multiple_of(wid * EP + i * C, 8)
            pltpu.sync_copy(src_hbm.at[pl.ds(base, C)], src_v)
            pltpu.sync_copy(dst_hbm.at[pl.ds(base, C)], dst_v)
            pltpu.async_copy(m_hbm.at[src_v], rows_v, sem).wait()
            pltpu.sync_copy(rows_v, acc.at[dst_v], add=True)
            if with_deg:
                pltpu.sync_copy(ones_v, dacc.at[dst_v], add=True)
            return carry

        lax.fori_loop(0, NCH, step, 0)
        plsc.subcore_barrier()

        @pl.when(sid < _NS - 1)
        def _():
            pltpu.sync_copy(acc.at[pl.ds(r0, RP)], agg_hbm.at[cid, pl.ds(r0, RP)])

        @pl.when(sid == _NS - 1)
        def _():
            t0 = (_NS - 1) * RP
            pltpu.sync_copy(
                acc.at[pl.ds(t0, RP_LAST)], agg_hbm.at[cid, pl.ds(t0, RP_LAST)]
            )
        if with_deg:

            @pl.when(sid == 0)
            def _():
                pltpu.sync_copy(dacc, deg_hbm.at[cid])

    return pl.kernel(
        body,
        out_type=out_type,
        mesh=mesh,
        scratch_types=scratch,
        compiler_params=pltpu.CompilerParams(use_tc_tiling_on_sc=False),
    )


def _mm1_body(x_ref, w_ref, b_ref, o_ref):
    o_ref[...] = (
        jnp.dot(x_ref[...], w_ref[...], preferred_element_type=jnp.float32)
        + b_ref[...]
    )


def _norm_from_partials(p_ref, deg_ref):
    agg = p_ref[0] + p_ref[1]
    deg = deg_ref[0] + deg_ref[1]
    h = jnp.maximum(agg / jnp.maximum(deg, 1.0), 0.0)
    nrm = jnp.sqrt(jnp.sum(h * h, axis=1, keepdims=True))
    return h / (nrm + 1e-6)


def _layer_body(p_ref, deg_ref, w_ref, b_ref, o_ref):
    h = _norm_from_partials(p_ref, deg_ref)
    o_ref[...] = (
        jnp.dot(h, w_ref[...], preferred_element_type=jnp.float32) + b_ref[...]
    )


def _head_body(p_ref, deg_ref, w1_ref, b1_ref, w2_ref, b2_ref, o_ref):
    h = _norm_from_partials(p_ref, deg_ref)
    z = jnp.maximum(
        jnp.dot(h, w1_ref[...], preferred_element_type=jnp.float32) + b1_ref[...],
        0.0,
    )
    y = jnp.dot(z, w2_ref[...], preferred_element_type=jnp.float32) + b2_ref[...]
    o_ref[...] = 1.0 / (1.0 + jnp.exp(-y))


def _mm1(x, W, b, bm=2000):
    N, D = x.shape
    H = W.shape[1]
    return pl.pallas_call(
        _mm1_body,
        grid=(N // bm,),
        in_specs=[
            pl.BlockSpec((bm, D), lambda i: (i, 0)),
            pl.BlockSpec((D, H), lambda i: (0, 0)),
            pl.BlockSpec((1, H), lambda i: (0, 0)),
        ],
        out_specs=pl.BlockSpec((bm, H), lambda i: (i, 0)),
        out_shape=jax.ShapeDtypeStruct((N, H), jnp.float32),
    )(x, W, b.reshape(1, H))


def _layer(p, deg, W, b, bm=2000):
    _, N, H = p.shape
    return pl.pallas_call(
        _layer_body,
        grid=(N // bm,),
        in_specs=[
            pl.BlockSpec((_NC, bm, H), lambda i: (0, i, 0)),
            pl.BlockSpec((_NC, bm, 1), lambda i: (0, i, 0)),
            pl.BlockSpec((H, H), lambda i: (0, 0)),
            pl.BlockSpec((1, H), lambda i: (0, 0)),
        ],
        out_specs=pl.BlockSpec((bm, H), lambda i: (i, 0)),
        out_shape=jax.ShapeDtypeStruct((N, H), jnp.float32),
    )(p, deg, W, b.reshape(1, H))


def _head(p, deg, Wm1, bm1, Wm2, bm2, bm=2000):
    _, N, H = p.shape
    K = Wm1.shape[1]
    return pl.pallas_call(
        _head_body,
        grid=(N // bm,),
        in_specs=[
            pl.BlockSpec((_NC, bm, H), lambda i: (0, i, 0)),
            pl.BlockSpec((_NC, bm, 1), lambda i: (0, i, 0)),
            pl.BlockSpec((H, K), lambda i: (0, 0)),
            pl.BlockSpec((1, K), lambda i: (0, 0)),
            pl.BlockSpec((K, 1), lambda i: (0, 0)),
            pl.BlockSpec((1, 1), lambda i: (0, 0)),
        ],
        out_specs=pl.BlockSpec((bm, 1), lambda i: (i, 0)),
        out_shape=jax.ShapeDtypeStruct((N, 1), jnp.float32),
    )(p, deg, Wm1, bm1.reshape(1, K), Wm2, bm2.reshape(1, 1))


def kernel(x, edge_index, W1, b1, W2, b2, W3, b3, Wm1, bm1, Wm2, bm2):
    N, D = x.shape
    H = W1.shape[1]
    E = edge_index.shape[1]
    src = edge_index[0]
    dst = edge_index[1]
    znh = jnp.zeros((N, H), jnp.float32)
    zn = jnp.zeros((N,), jnp.float32)

    agg_first = _make_aggregate(N, H, E, with_deg=True)
    agg_rest = _make_aggregate(N, H, E, with_deg=False)

    m1 = _mm1(x, W1, b1)
    p1, deg2 = agg_first(m1, src, dst, znh, zn)
    deg2 = deg2.reshape(_NC, N, 1)
    m2 = _layer(p1, deg2, W2, b2)
    (p2,) = agg_rest(m2, src, dst, znh)
    m3 = _layer(p2, deg2, W3, b3)
    (p3,) = agg_rest(m3, src, dst, znh)
    return _head(p3, deg2, Wm1, bm1, Wm2, bm2)


# speedup vs baseline: 5.6101x; 1.0870x over previous
import functools

import jax
import jax.numpy as jnp
from jax import lax
from jax.experimental import pallas as pl
from jax.experimental.pallas import tpu as pltpu
from jax.experimental.pallas import tpu_sc as plsc

_NC = 2
_NS = 16


_C = 128


def _make_aggregate(N, H, E_pad, with_deg):
    NW = _NC * _NS
    NCH = E_pad // (_C * NW)
    assert NCH * _C * NW == E_pad and NCH % 2 == 0
    NP = NCH // 2
    NA = N + 8
    RP = (-(-NA // _NS) + 7) // 8 * 8
    Z_LAST = NA - (_NS - 1) * RP
    O_LAST = N - (_NS - 1) * RP
    assert Z_LAST > 0 and Z_LAST % 8 == 0 and O_LAST > 0 and O_LAST % 8 == 0

    mesh = plsc.VectorSubcoreMesh(core_axis_name="c", subcore_axis_name="s")

    out_type = [jax.ShapeDtypeStruct((_NC, N, H), jnp.float32)]
    scratch = [
        pltpu.VMEM((NCH, _C), jnp.int32),
        pltpu.VMEM((NCH, _C), jnp.int32),
        pltpu.VMEM((_C, H), jnp.float32),
        pltpu.VMEM((_C, H), jnp.float32),
        pltpu.VMEM_SHARED((NA, H), jnp.float32),
        pltpu.SemaphoreType.DMA,
        pltpu.SemaphoreType.DMA,
    ]
    if with_deg:
        out_type.append(jax.ShapeDtypeStruct((_NC, N), jnp.float32))
        scratch += [
            pltpu.VMEM((_C,), jnp.float32),
            pltpu.VMEM_SHARED((NA,), jnp.float32),
        ]

    def body(*refs):
        if with_deg:
            (m_hbm, src_hbm, dst_hbm, znh_hbm, zn_hbm,
             agg_hbm, deg_hbm, src2d, dst2d, bufa, bufb, acc, sema, semb,
             ones_v, dacc) = refs
        else:
            (m_hbm, src_hbm, dst_hbm, znh_hbm,
             agg_hbm, src2d, dst2d, bufa, bufb, acc, sema, semb) = refs
        cid = lax.axis_index("c")
        sid = lax.axis_index("s")
        wid = sid * _NC + cid

        c0 = pl.multiple_of(wid * NCH, 8)
        pltpu.sync_copy(src_hbm.at[pl.ds(c0, NCH)], src2d)
        pltpu.sync_copy(dst_hbm.at[pl.ds(c0, NCH)], dst2d)

        r0 = pl.multiple_of(sid * RP, 8)

        @pl.when(sid < _NS - 1)
        def _():
            pltpu.sync_copy(znh_hbm.at[pl.ds(r0, RP)], acc.at[pl.ds(r0, RP)])

        @pl.when(sid == _NS - 1)
        def _():
            t0 = (_NS - 1) * RP
            pltpu.sync_copy(
                znh_hbm.at[pl.ds(t0, Z_LAST)], acc.at[pl.ds(t0, Z_LAST)]
            )
        if with_deg:

            @pl.when(sid == 0)
            def _():
                pltpu.sync_copy(zn_hbm, dacc)

            for j in range(_C // 16):
                ones_v[pl.ds(j * 16, 16)] = jnp.ones((16,), jnp.float32)
        plsc.subcore_barrier()

        def gfire(i, buf, sem):
            pltpu.async_copy(m_hbm.at[src2d.at[i]], buf, sem)

        def gwait(buf, sem):
            pltpu.make_async_copy(m_hbm.at[src2d.at[0]], buf, sem).wait()

        def scat(i, buf):
            pltpu.sync_copy(buf, acc.at[dst2d.at[i]], add=True)
            if with_deg:
                pltpu.sync_copy(ones_v, dacc.at[dst2d.at[i]], add=True)

        gfire(0, bufa, sema)
        gfire(1, bufb, semb)

        def pair(p, carry):
            i = p * 2
            gwait(bufa, sema)
            scat(i, bufa)

            @pl.when(p < NP - 1)
            def _():
                gfire(i + 2, bufa, sema)

            gwait(bufb, semb)
            scat(i + 1, bufb)

            @pl.when(p < NP - 1)
            def _():
                gfire(i + 3, bufb, semb)

            return carry

        lax.fori_loop(0, NP, pair, 0)
        plsc.subcore_barrier()

        @pl.when(sid < _NS - 1)
        def _():
            pltpu.sync_copy(acc.at[pl.ds(r0, RP)], agg_hbm.at[cid, pl.ds(r0, RP)])

        @pl.when(sid == _NS - 1)
        def _():
            t0 = (_NS - 1) * RP
            pltpu.sync_copy(
                acc.at[pl.ds(t0, O_LAST)], agg_hbm.at[cid, pl.ds(t0, O_LAST)]
            )
        if with_deg:

            @pl.when(sid == 0)
            def _():
                pltpu.sync_copy(dacc.at[pl.ds(0, N)], deg_hbm.at[cid])

    return pl.kernel(
        body,
        out_type=out_type,
        mesh=mesh,
        scratch_types=scratch,
        compiler_params=pltpu.CompilerParams(use_tc_tiling_on_sc=False),
    )


def _mm1_body(x_ref, w_ref, b_ref, o_ref):
    o_ref[...] = (
        jnp.dot(x_ref[...], w_ref[...], preferred_element_type=jnp.float32)
        + b_ref[...]
    )


def _norm_from_partials(p_ref, deg_ref):
    agg = p_ref[0] + p_ref[1]
    deg = deg_ref[0] + deg_ref[1]
    h = jnp.maximum(agg / jnp.maximum(deg, 1.0), 0.0)
    nrm = jnp.sqrt(jnp.sum(h * h, axis=1, keepdims=True))
    return h / (nrm + 1e-6)


def _layer_body(p_ref, deg_ref, w_ref, b_ref, o_ref):
    h = _norm_from_partials(p_ref, deg_ref)
    o_ref[...] = (
        jnp.dot(h, w_ref[...], preferred_element_type=jnp.float32) + b_ref[...]
    )


def _head_body(p_ref, deg_ref, w1_ref, b1_ref, w2_ref, b2_ref, o_ref):
    h = _norm_from_partials(p_ref, deg_ref)
    z = jnp.maximum(
        jnp.dot(h, w1_ref[...], preferred_element_type=jnp.float32) + b1_ref[...],
        0.0,
    )
    y = jnp.dot(z, w2_ref[...], preferred_element_type=jnp.float32) + b2_ref[...]
    o_ref[...] = 1.0 / (1.0 + jnp.exp(-y))


def _mm1(x, W, b, bm=2000):
    N, D = x.shape
    H = W.shape[1]
    return pl.pallas_call(
        _mm1_body,
        grid=(N // bm,),
        in_specs=[
            pl.BlockSpec((bm, D), lambda i: (i, 0)),
            pl.BlockSpec((D, H), lambda i: (0, 0)),
            pl.BlockSpec((1, H), lambda i: (0, 0)),
        ],
        out_specs=pl.BlockSpec((bm, H), lambda i: (i, 0)),
        out_shape=jax.ShapeDtypeStruct((N, H), jnp.float32),
    )(x, W, b.reshape(1, H))


def _layer(p, deg, W, b, bm=2000):
    _, N, H = p.shape
    return pl.pallas_call(
        _layer_body,
        grid=(N // bm,),
        in_specs=[
            pl.BlockSpec((_NC, bm, H), lambda i: (0, i, 0)),
            pl.BlockSpec((_NC, bm, 1), lambda i: (0, i, 0)),
            pl.BlockSpec((H, H), lambda i: (0, 0)),
            pl.BlockSpec((1, H), lambda i: (0, 0)),
        ],
        out_specs=pl.BlockSpec((bm, H), lambda i: (i, 0)),
        out_shape=jax.ShapeDtypeStruct((N, H), jnp.float32),
    )(p, deg, W, b.reshape(1, H))


def _head(p, deg, Wm1, bm1, Wm2, bm2, bm=2000):
    _, N, H = p.shape
    K = Wm1.shape[1]
    return pl.pallas_call(
        _head_body,
        grid=(N // bm,),
        in_specs=[
            pl.BlockSpec((_NC, bm, H), lambda i: (0, i, 0)),
            pl.BlockSpec((_NC, bm, 1), lambda i: (0, i, 0)),
            pl.BlockSpec((H, K), lambda i: (0, 0)),
            pl.BlockSpec((1, K), lambda i: (0, 0)),
            pl.BlockSpec((K, 1), lambda i: (0, 0)),
            pl.BlockSpec((1, 1), lambda i: (0, 0)),
        ],
        out_specs=pl.BlockSpec((bm, 1), lambda i: (i, 0)),
        out_shape=jax.ShapeDtypeStruct((N, 1), jnp.float32),
    )(p, deg, Wm1, bm1.reshape(1, K), Wm2, bm2.reshape(1, 1))


def kernel(x, edge_index, W1, b1, W2, b2, W3, b3, Wm1, bm1, Wm2, bm2):
    N, D = x.shape
    H = W1.shape[1]
    E = edge_index.shape[1]
    NW = _NC * _NS
    ncw = -(-E // (_C * NW))
    ncw += ncw % 2
    E_pad = ncw * _C * NW
    src = jnp.concatenate(
        [edge_index[0], jnp.zeros((E_pad - E,), jnp.int32)]
    ).reshape(-1, _C)
    dst = jnp.concatenate(
        [edge_index[1], jnp.full((E_pad - E,), N, jnp.int32)]
    ).reshape(-1, _C)
    znh = jnp.zeros((N + 8, H), jnp.float32)
    zn = jnp.zeros((N + 8,), jnp.float32)

    agg_first = _make_aggregate(N, H, E_pad, with_deg=True)
    agg_rest = _make_aggregate(N, H, E_pad, with_deg=False)

    m1 = _mm1(x, W1, b1)
    p1, deg2 = agg_first(m1, src, dst, znh, zn)
    deg2 = deg2.reshape(_NC, N, 1)
    m2 = _layer(p1, deg2, W2, b2)
    (p2,) = agg_rest(m2, src, dst, znh)
    m3 = _layer(p2, deg2, W3, b3)
    (p3,) = agg_rest(m3, src, dst, znh)
    return _head(p3, deg2, Wm1, bm1, Wm2, bm2)
